# baseline (speedup 1.0000x reference)
import jax
import jax.numpy as jnp
from jax.experimental import pallas as pl

TOPK = 100
SBLK = 64
CBLK = 512


def _score_body(q_ref, v_ref, wc_ref, w2c_ref, b12_ref, vblk_ref, bv_ref,
                wo_ref, bo_ref, out_ref):
    nb, sblk, h = v_ref.shape
    ku = wc_ref.shape[1]
    qb = jax.lax.dot_general(q_ref[...].astype(jnp.bfloat16), w2c_ref[...],
                             (((1,), (0,)), ((), ())),
                             preferred_element_type=jnp.float32)
    v2 = v_ref[...].reshape(nb * sblk, h).astype(jnp.bfloat16)
    s1 = jax.lax.dot_general(v2, wc_ref[...],
                             (((1,), (0,)), ((), ())),
                             preferred_element_type=jnp.float32)
    s1 = s1.reshape(nb, sblk, ku)
    b1r = b12_ref[0:1, :]
    b2r = b12_ref[1:2, :]
    s1 = (s1 + b1r[:, None, :]) + (qb + b2r)[:, None, :]
    t = jnp.tanh(s1)
    tb = t.astype(jnp.bfloat16).reshape(nb * sblk, ku)
    sck = jax.lax.dot_general(tb, vblk_ref[...],
                              (((1,), (0,)), ((), ())),
                              preferred_element_type=jnp.float32)
    sck = sck + bv_ref[...]
    sckb = sck.astype(jnp.bfloat16)
    score = jax.lax.dot_general(sckb, wo_ref[...],
                                (((1,), (0,)), ((), ())),
                                preferred_element_type=jnp.float32)
    out_ref[...] = score + bo_ref[0, 0]


def _topk_body(sc_ref, m_ref, aw_ref):
    x = sc_ref[...]
    bits = jax.lax.bitcast_convert_type(x, jnp.int32)
    neg = jnp.int32(-2147483648) - bits - 1
    s = jnp.where(bits >= 0, bits, neg)
    nb, ns = x.shape

    def bs_body(_, lohi):
        lo, hi = lohi
        mid = (lo >> 1) + (hi >> 1) + ((lo | hi) & 1)
        cnt = jnp.sum((s >= mid).astype(jnp.int32), axis=1, keepdims=True)
        ge = cnt >= TOPK
        return jnp.where(ge, mid, lo), jnp.where(ge, hi, mid - 1)

    lo0 = jnp.full((nb, 1), -2**31, jnp.int32)
    hi0 = jnp.full((nb, 1), 2**31 - 1, jnp.int32)
    thr, _ = jax.lax.fori_loop(0, 32, bs_body, (lo0, hi0))

    gt = s > thr
    eq = s == thr
    cnt_gt = jnp.sum(gt.astype(jnp.int32), axis=1, keepdims=True)
    need = TOPK - cnt_gt
    idx = jax.lax.broadcasted_iota(jnp.int32, (nb, ns), 1)

    def bs2_body(_, lohi):
        lo, hi = lohi
        mid = (lo + hi) >> 1
        cnt = jnp.sum((eq & (idx < mid)).astype(jnp.int32), axis=1,
                      keepdims=True)
        ge = cnt >= need
        return jnp.where(ge, lo, mid + 1), jnp.where(ge, mid, hi)

    lo0 = jnp.zeros((nb, 1), jnp.int32)
    hi0 = jnp.full((nb, 1), ns, jnp.int32)
    istar, _ = jax.lax.fori_loop(0, 13, bs2_body, (lo0, hi0))

    mask = gt | (eq & (idx < istar))
    m = jnp.where(mask, x, 0.0)
    sig = 1.0 / (1.0 + jnp.exp(-m))
    denom = jnp.sum(sig, axis=0, keepdims=True)
    m_ref[...] = m
    aw_ref[...] = sig / denom


def _ctx_body(aw_ref, v_ref, out_ref):
    j = pl.program_id(0)
    nb = aw_ref.shape[0]

    @pl.when(j == 0)
    def _():
        out_ref[...] = jnp.zeros_like(out_ref)

    for b in range(nb):
        part = jax.lax.dot_general(
            aw_ref[b:b + 1, :], v_ref[b], (((1,), (0,)), ((), ())),
            preferred_element_type=jnp.float32)
        out_ref[b:b + 1, :] += part


def _compute_scores(query, values, W1, b1, W2, b2, V, bV, Wo, bo):
    B, S, H = values.shape
    K, _, U = W1.shape
    KU = K * U

    wc = W1.transpose(1, 0, 2).reshape(H, KU).astype(jnp.bfloat16)
    w2c = W2.transpose(1, 0, 2).reshape(H, KU).astype(jnp.bfloat16)
    b12 = jnp.concatenate([b1.reshape(1, KU), b2.reshape(1, KU)], axis=0)
    v16 = V.astype(jnp.bfloat16)
    vblk = (v16[:, :, None] *
            jnp.eye(K, dtype=jnp.bfloat16)[:, None, :]).reshape(KU, K)
    bvr = bV.reshape(1, K)
    woc = Wo.reshape(K, 1).astype(jnp.bfloat16)
    bor = bo.reshape(1, 1)

    ns = S // SBLK
    nrow = B * SBLK
    flat = pl.pallas_call(
        _score_body,
        grid=(ns,),
        in_specs=[
            pl.BlockSpec((B, H), lambda j: (0, 0)),
            pl.BlockSpec((B, SBLK, H), lambda j: (0, j, 0)),
            pl.BlockSpec((H, KU), lambda j: (0, 0)),
            pl.BlockSpec((H, KU), lambda j: (0, 0)),
            pl.BlockSpec((2, KU), lambda j: (0, 0)),
            pl.BlockSpec((KU, K), lambda j: (0, 0)),
            pl.BlockSpec((1, K), lambda j: (0, 0)),
            pl.BlockSpec((K, 1), lambda j: (0, 0)),
            pl.BlockSpec((1, 1), lambda j: (0, 0)),
        ],
        out_specs=pl.BlockSpec((nrow, 1), lambda j: (j, 0)),
        out_shape=jax.ShapeDtypeStruct((ns * nrow, 1), jnp.float32),
    )(query, values, wc, w2c, b12, vblk, bvr, woc, bor)
    score = flat.reshape(ns, B, SBLK).transpose(1, 0, 2).reshape(B, S)
    return score


def kernel(query, values, W1, b1, W2, b2, V, bV, Wo, bo):
    B, S, H = values.shape
    score = _compute_scores(query, values, W1, b1, W2, b2, V, bV, Wo, bo)

    m, aw = pl.pallas_call(
        _topk_body,
        out_shape=(jax.ShapeDtypeStruct((B, S), jnp.float32),
                   jax.ShapeDtypeStruct((B, S), jnp.float32)),
    )(score)

    K, _, U = W1.shape
    nc = S // CBLK
    context = pl.pallas_call(
        _ctx_body,
        grid=(nc,),
        in_specs=[
            pl.BlockSpec((B, CBLK), lambda j: (0, j)),
            pl.BlockSpec((B, CBLK, H), lambda j: (0, j, 0)),
        ],
        out_specs=pl.BlockSpec((B, H), lambda j: (0, 0)),
        out_shape=jax.ShapeDtypeStruct((B, H), jnp.float32),
    )(aw, values)

    return (context, aw[..., None], m[..., None])


# speedup vs baseline: 1.1245x; 1.1245x over previous
import jax
import jax.numpy as jnp
from jax.experimental import pallas as pl

TOPK = 100
SBLK = 128
CBLK = 1024


def _score_body(q_ref, v_ref, wc_ref, w2c_ref, b12_ref, vblk_ref, bv_ref,
                wo_ref, bo_ref, out_ref):
    nb, sblk, h = v_ref.shape
    ku = wc_ref.shape[1]
    qb = jax.lax.dot_general(q_ref[...].astype(jnp.bfloat16), w2c_ref[...],
                             (((1,), (0,)), ((), ())),
                             preferred_element_type=jnp.float32)
    v2 = v_ref[...].reshape(nb * sblk, h).astype(jnp.bfloat16)
    s1 = jax.lax.dot_general(v2, wc_ref[...],
                             (((1,), (0,)), ((), ())),
                             preferred_element_type=jnp.float32)
    s1 = s1.reshape(nb, sblk, ku)
    b1r = b12_ref[0:1, :]
    b2r = b12_ref[1:2, :]
    s1 = (s1 + b1r[:, None, :]) + (qb + b2r)[:, None, :]
    t = jnp.tanh(s1)
    tb = t.astype(jnp.bfloat16).reshape(nb * sblk, ku)
    sck = jax.lax.dot_general(tb, vblk_ref[...],
                              (((1,), (0,)), ((), ())),
                              preferred_element_type=jnp.float32)
    sck = sck + bv_ref[...]
    sckb = sck.astype(jnp.bfloat16)
    score = jax.lax.dot_general(sckb, wo_ref[...],
                                (((1,), (0,)), ((), ())),
                                preferred_element_type=jnp.float32)
    out_ref[...] = score + bo_ref[0, 0]


def _topk_body(sc_ref, m_ref, aw_ref):
    x = sc_ref[...]
    bits = jax.lax.bitcast_convert_type(x, jnp.int32)
    neg = jnp.int32(-2147483648) - bits - 1
    s = jnp.where(bits >= 0, bits, neg)
    nb, ns = x.shape

    def bs_body(_, lohi):
        lo, hi = lohi
        mid = (lo >> 1) + (hi >> 1) + ((lo | hi) & 1)
        cnt = jnp.sum((s >= mid).astype(jnp.int32), axis=1, keepdims=True)
        ge = cnt >= TOPK
        return jnp.where(ge, mid, lo), jnp.where(ge, hi, mid - 1)

    lo0 = jnp.full((nb, 1), -2**31, jnp.int32)
    hi0 = jnp.full((nb, 1), 2**31 - 1, jnp.int32)
    thr, _ = jax.lax.fori_loop(0, 32, bs_body, (lo0, hi0))

    gt = s > thr
    eq = s == thr
    cnt_gt = jnp.sum(gt.astype(jnp.int32), axis=1, keepdims=True)
    need = TOPK - cnt_gt
    idx = jax.lax.broadcasted_iota(jnp.int32, (nb, ns), 1)

    def bs2_body(_, lohi):
        lo, hi = lohi
        mid = (lo + hi) >> 1
        cnt = jnp.sum((eq & (idx < mid)).astype(jnp.int32), axis=1,
                      keepdims=True)
        ge = cnt >= need
        return jnp.where(ge, lo, mid + 1), jnp.where(ge, mid, hi)

    lo0 = jnp.zeros((nb, 1), jnp.int32)
    hi0 = jnp.full((nb, 1), ns, jnp.int32)
    istar, _ = jax.lax.fori_loop(0, 13, bs2_body, (lo0, hi0))

    mask = gt | (eq & (idx < istar))
    m = jnp.where(mask, x, 0.0)
    sig = 1.0 / (1.0 + jnp.exp(-m))
    denom = jnp.sum(sig, axis=0, keepdims=True)
    m_ref[...] = m
    aw_ref[...] = sig / denom


def _ctx_body(aw_ref, v_ref, out_ref):
    j = pl.program_id(0)
    nb = aw_ref.shape[0]

    @pl.when(j == 0)
    def _():
        out_ref[...] = jnp.zeros_like(out_ref)

    for b in range(nb):
        part = jax.lax.dot_general(
            aw_ref[b:b + 1, :], v_ref[b], (((1,), (0,)), ((), ())),
            preferred_element_type=jnp.float32)
        out_ref[b:b + 1, :] += part


def _compute_scores(query, values, W1, b1, W2, b2, V, bV, Wo, bo):
    B, S, H = values.shape
    K, _, U = W1.shape
    KU = K * U

    wc = W1.transpose(1, 0, 2).reshape(H, KU).astype(jnp.bfloat16)
    w2c = W2.transpose(1, 0, 2).reshape(H, KU).astype(jnp.bfloat16)
    b12 = jnp.concatenate([b1.reshape(1, KU), b2.reshape(1, KU)], axis=0)
    v16 = V.astype(jnp.bfloat16)
    vblk = (v16[:, :, None] *
            jnp.eye(K, dtype=jnp.bfloat16)[:, None, :]).reshape(KU, K)
    bvr = bV.reshape(1, K)
    woc = Wo.reshape(K, 1).astype(jnp.bfloat16)
    bor = bo.reshape(1, 1)

    ns = S // SBLK
    nrow = B * SBLK
    flat = pl.pallas_call(
        _score_body,
        grid=(ns,),
        in_specs=[
            pl.BlockSpec((B, H), lambda j: (0, 0)),
            pl.BlockSpec((B, SBLK, H), lambda j: (0, j, 0)),
            pl.BlockSpec((H, KU), lambda j: (0, 0)),
            pl.BlockSpec((H, KU), lambda j: (0, 0)),
            pl.BlockSpec((2, KU), lambda j: (0, 0)),
            pl.BlockSpec((KU, K), lambda j: (0, 0)),
            pl.BlockSpec((1, K), lambda j: (0, 0)),
            pl.BlockSpec((K, 1), lambda j: (0, 0)),
            pl.BlockSpec((1, 1), lambda j: (0, 0)),
        ],
        out_specs=pl.BlockSpec((nrow, 1), lambda j: (j, 0)),
        out_shape=jax.ShapeDtypeStruct((ns * nrow, 1), jnp.float32),
    )(query, values, wc, w2c, b12, vblk, bvr, woc, bor)
    score = flat.reshape(ns, B, SBLK).transpose(1, 0, 2).reshape(B, S)
    return score


def kernel(query, values, W1, b1, W2, b2, V, bV, Wo, bo):
    B, S, H = values.shape
    score = _compute_scores(query, values, W1, b1, W2, b2, V, bV, Wo, bo)

    m, aw = pl.pallas_call(
        _topk_body,
        out_shape=(jax.ShapeDtypeStruct((B, S), jnp.float32),
                   jax.ShapeDtypeStruct((B, S), jnp.float32)),
    )(score)

    K, _, U = W1.shape
    nc = S // CBLK
    context = pl.pallas_call(
        _ctx_body,
        grid=(nc,),
        in_specs=[
            pl.BlockSpec((B, CBLK), lambda j: (0, j)),
            pl.BlockSpec((B, CBLK, H), lambda j: (0, j, 0)),
        ],
        out_specs=pl.BlockSpec((B, H), lambda j: (0, 0)),
        out_shape=jax.ShapeDtypeStruct((B, H), jnp.float32),
    )(aw, values)

    return (context, aw[..., None], m[..., None])
